# Initial kernel scaffold; baseline (speedup 1.0000x reference)
import jax
import jax.numpy as jnp
from jax.experimental import pallas as pl


def kernel(user_emb, item_emb, edge_val, edge_src, edge_dst):
    raise NotImplementedError("write your pallas kernel here")


# speedup vs baseline: 4.0738x; 4.0738x over previous
import jax
import jax.numpy as jnp
from jax import lax
from jax.experimental import pallas as pl
from jax.experimental.pallas import tpu as pltpu
from jax.experimental.pallas import tpu_sc as plsc

N_USERS = 25000
N_ITEMS = 25000
N = N_USERS + N_ITEMS
N_PAD = 51200
H = 32
NC, NS = 2, 16
SUB = 128
SUBS_PER_BLOCK = 16
BLOCK = SUB * SUBS_PER_BLOCK
BLOCKS_PER_TILE = 25
EDGES_PER_TILE = BLOCK * BLOCKS_PER_TILE
E_PAD = EDGES_PER_TILE * NS
ROWS_PER_TILE = N_PAD // NS
ZCH = 128
NZC = ROWS_PER_TILE // ZCH


def _body(ego_hbm, src_hbm, dst_hbm, val_hbm, out_hbm,
          t1, t2, t3, acc, srcb, dstb, valb, idxb, rows, zbuf,
          a1, a2, a3, sem):
    c = lax.axis_index("c")
    s = lax.axis_index("s")

    def zrow(i, _):
        zbuf[i, pl.ds(0, 16)] = jnp.zeros((16,), jnp.float32)
        zbuf[i, pl.ds(16, 16)] = jnp.zeros((16,), jnp.float32)
        return 0
    lax.fori_loop(0, ZCH, zrow, 0)

    def layer(src_tab, dst_tab):
        def zc(z, _):
            pltpu.sync_copy(zbuf, acc.at[pl.ds(s * ROWS_PER_TILE + z * ZCH, ZCH)])
            return 0
        lax.fori_loop(0, NZC, zc, 0)
        plsc.subcore_barrier()

        def blk(b, _):
            row0 = s * (EDGES_PER_TILE // SUB) + b * SUBS_PER_BLOCK
            pltpu.sync_copy(src_hbm.at[pl.ds(row0, SUBS_PER_BLOCK)], srcb)
            pltpu.sync_copy(dst_hbm.at[pl.ds(row0, SUBS_PER_BLOCK)], dstb)
            pltpu.sync_copy(val_hbm.at[pl.ds(row0, SUBS_PER_BLOCK)], valb)

            def sub(j, _):
                def mkidx(u, _):
                    idxb[pl.ds(u * 16, 16)] = srcb[j, pl.ds(u * 16, 16)] + c * N_PAD
                    return 0
                lax.fori_loop(0, SUB // 16, mkidx, 0)
                pltpu.async_copy(src_tab.at[idxb], rows, sem).wait()

                def mulg(g, _):
                    vv = valb[j, pl.ds(g * 16, 16)]
                    for l in range(16):
                        e = g * 16 + l
                        v = vv[l]
                        rows[e, pl.ds(0, 16)] = rows[e, pl.ds(0, 16)] * v
                        rows[e, pl.ds(16, 16)] = rows[e, pl.ds(16, 16)] * v
                    return 0
                lax.fori_loop(0, SUB // 16, mulg, 0)

                pltpu.sync_copy(rows, acc.at[dstb.at[j]], add=True)
                return 0
            lax.fori_loop(0, SUBS_PER_BLOCK, sub, 0)
            return 0
        lax.fori_loop(0, BLOCKS_PER_TILE, blk, 0)
        plsc.subcore_barrier()

        pltpu.sync_copy(
            acc.at[pl.ds(s * ROWS_PER_TILE, ROWS_PER_TILE)],
            dst_tab.at[pl.ds(c * N_PAD + s * ROWS_PER_TILE, ROWS_PER_TILE)])
        plsc.subcore_barrier()

    layer(ego_hbm, t1)
    layer(t1, t2)
    layer(t2, t3)

    def fin(z, _):
        r0 = c * N_PAD + s * ROWS_PER_TILE + z * ZCH
        pltpu.sync_copy(t1.at[pl.ds(r0, ZCH)], a1)
        pltpu.sync_copy(t2.at[pl.ds(r0, ZCH)], a2)
        pltpu.sync_copy(t3.at[pl.ds(r0, ZCH)], a3)

        def m(i, _):
            for o in (0, 16):
                x = (a1[i, pl.ds(o, 16)] + a2[i, pl.ds(o, 16)]
                     + a3[i, pl.ds(o, 16)]) * jnp.float32(1.0 / 3.0)
                a1[i, pl.ds(o, 16)] = x
            return 0
        lax.fori_loop(0, ZCH, m, 0)
        pltpu.sync_copy(a1, out_hbm.at[pl.ds(r0, ZCH)])
        return 0
    lax.fori_loop(0, NZC, fin, 0)


_run = pl.kernel(
    _body,
    out_type=jax.ShapeDtypeStruct((2 * N_PAD, H), jnp.float32),
    mesh=plsc.VectorSubcoreMesh(core_axis_name="c", subcore_axis_name="s"),
    compiler_params=pltpu.CompilerParams(use_tc_tiling_on_sc=False),
    scratch_types=[
        pltpu.HBM((2 * N_PAD, H), jnp.float32),
        pltpu.HBM((2 * N_PAD, H), jnp.float32),
        pltpu.HBM((2 * N_PAD, H), jnp.float32),
        pltpu.VMEM_SHARED((N_PAD, H), jnp.float32),
        pltpu.VMEM((SUBS_PER_BLOCK, SUB), jnp.int32),
        pltpu.VMEM((SUBS_PER_BLOCK, SUB), jnp.int32),
        pltpu.VMEM((SUBS_PER_BLOCK, SUB), jnp.float32),
        pltpu.VMEM((SUB,), jnp.int32),
        pltpu.VMEM((SUB, H), jnp.float32),
        pltpu.VMEM((ZCH, H), jnp.float32),
        pltpu.VMEM((ZCH, H), jnp.float32),
        pltpu.VMEM((ZCH, H), jnp.float32),
        pltpu.VMEM((ZCH, H), jnp.float32),
        pltpu.SemaphoreType.DMA,
    ],
)


def kernel(user_emb, item_emb, edge_val, edge_src, edge_dst):
    ego = jnp.concatenate([user_emb, item_emb], axis=0)
    rpad = jnp.zeros((N_PAD - N, H), jnp.float32)
    ego_flat = jnp.concatenate(
        [ego[:, :H], rpad, ego[:, H:], rpad], axis=0)
    e = edge_src.shape[0]
    pad = E_PAD - e
    src = jnp.pad(edge_src.astype(jnp.int32), (0, pad)).reshape(E_PAD // SUB, SUB)
    dst = jnp.pad(edge_dst.astype(jnp.int32), (0, pad)).reshape(E_PAD // SUB, SUB)
    val = jnp.pad(edge_val, (0, pad)).reshape(E_PAD // SUB, SUB)
    out = _run(ego_flat, src, dst, val)
    final = jnp.concatenate([out[:N], out[N_PAD:N_PAD + N]], axis=1)
    return final[:N_USERS], final[N_USERS:]


# speedup vs baseline: 7.0419x; 1.7286x over previous
import jax
import jax.numpy as jnp
from jax import lax
from jax.experimental import pallas as pl
from jax.experimental.pallas import tpu as pltpu
from jax.experimental.pallas import tpu_sc as plsc

N_USERS = 25000
N_ITEMS = 25000
N = N_USERS + N_ITEMS
N_PAD = 51200
H = 32
NC, NS = 2, 16
SUB = 128
GP = 4
CPT = 400
NG = CPT // GP
EDGES_PER_TILE = CPT * SUB
E_PAD = EDGES_PER_TILE * NS
ROWS_PER_TILE = N_PAD // NS
ZCH = 128
NZC = ROWS_PER_TILE // ZCH


def _body(ego_hbm, src_hbm, dst_hbm, val_hbm, out_hbm,
          t1, t2, t3, acc,
          sb0, db0, vb0, sb1, db1, vb1,
          i0, i1, i2, i3, x0, x1, x2, x3,
          r0, r1, r2, r3,
          gs0, gs1, gs2, gs3, ss0, ss1, ss2, ss3, esem):
    c = lax.axis_index("c")
    s = lax.axis_index("s")
    idxb = [i0, i1, i2, i3]
    dstx = [x0, x1, x2, x3]
    rows = [r0, r1, r2, r3]
    gsem = [gs0, gs1, gs2, gs3]
    ssem = [ss0, ss1, ss2, ss3]
    off = c * N_PAD
    tile0 = s * CPT

    def fill_zero(buf):
        def zr(i, _):
            buf[i, pl.ds(0, 16)] = jnp.zeros((16,), jnp.float32)
            buf[i, pl.ds(16, 16)] = jnp.zeros((16,), jnp.float32)
            return 0
        lax.fori_loop(0, ZCH, zr, 0)

    def zero_stripe():
        fill_zero(r3)

        def zc(z, _):
            pltpu.sync_copy(r3, acc.at[pl.ds(s * ROWS_PER_TILE + z * ZCH, ZCH)])
            return 0
        lax.fori_loop(0, NZC, zc, 0)

    def build_idx(rr, sb, prow):
        def mk(u, _):
            idxb[rr][pl.ds(u * 16, 16)] = sb[prow, pl.ds(u * 16, 16)] + off
            return 0
        lax.fori_loop(0, SUB // 16, mk, 0)

    def copy_dst(rr, db, prow):
        def mk(u, _):
            dstx[rr][pl.ds(u * 16, 16)] = db[prow, pl.ds(u * 16, 16)]
            return 0
        lax.fori_loop(0, SUB // 16, mk, 0)

    def mul_rows(rr, vb, prow):
        def mg(g8, _):
            vv = vb[prow, pl.ds(g8 * 16, 16)]
            for l in range(16):
                e = g8 * 16 + l
                v = vv[l]
                rows[rr][e, pl.ds(0, 16)] = rows[rr][e, pl.ds(0, 16)] * v
                rows[rr][e, pl.ds(16, 16)] = rows[rr][e, pl.ds(16, 16)] * v
            return 0
        lax.fori_loop(0, SUB // 16, mg, 0)

    def scatter_wait(rr):
        pltpu.make_async_copy(rows[rr], acc.at[dstx[rr]], ssem[rr]).wait()

    def edges(src_tab):
        pltpu.sync_copy(src_hbm.at[pl.ds(tile0, GP)], sb0)
        pltpu.sync_copy(dst_hbm.at[pl.ds(tile0, GP)], db0)
        pltpu.sync_copy(val_hbm.at[pl.ds(tile0, GP)], vb0)
        for p in (0, 1):
            build_idx(p, sb0, p)
            pltpu.async_copy(src_tab.at[idxb[p]], rows[p], gsem[p])

        def gg_body(gg, _):
            for h in (0, 1):
                g = 2 * gg + h
                sb, db, vb = (sb0, db0, vb0) if h == 0 else (sb1, db1, vb1)
                nsb, ndb, nvb = (sb1, db1, vb1) if h == 0 else (sb0, db0, vb0)
                nrow = tile0 + (g + 1) * GP

                @pl.when(g < NG - 1)
                def _stage():
                    pltpu.async_copy(src_hbm.at[pl.ds(nrow, GP)], nsb, esem)
                    pltpu.async_copy(dst_hbm.at[pl.ds(nrow, GP)], ndb, esem)
                    pltpu.async_copy(val_hbm.at[pl.ds(nrow, GP)], nvb, esem)

                for p in range(GP):
                    r2 = (p + 2) % 4
                    if p < 2:
                        @pl.when(g >= 1)
                        def _drain():
                            scatter_wait(r2)
                        build_idx(r2, sb, p + 2)
                        pltpu.async_copy(src_tab.at[idxb[r2]], rows[r2], gsem[r2])
                    else:
                        scatter_wait(r2)

                        @pl.when(g < NG - 1)
                        def _pref():
                            if p == 2:
                                pltpu.make_async_copy(
                                    src_hbm.at[pl.ds(nrow, GP)], nsb, esem).wait()
                                pltpu.make_async_copy(
                                    dst_hbm.at[pl.ds(nrow, GP)], ndb, esem).wait()
                                pltpu.make_async_copy(
                                    val_hbm.at[pl.ds(nrow, GP)], nvb, esem).wait()
                            build_idx(r2, nsb, p - 2)
                            pltpu.async_copy(src_tab.at[idxb[r2]], rows[r2], gsem[r2])

                    pltpu.make_async_copy(src_tab.at[idxb[p]], rows[p], gsem[p]).wait()
                    mul_rows(p, vb, p)
                    copy_dst(p, db, p)
                    pltpu.async_copy(rows[p], acc.at[dstx[p]], ssem[p], add=True)
            return 0
        lax.fori_loop(0, NG // 2, gg_body, 0)
        scatter_wait(2)
        scatter_wait(3)

    def writeback(dst_tab):
        pltpu.sync_copy(
            acc.at[pl.ds(s * ROWS_PER_TILE, ROWS_PER_TILE)],
            dst_tab.at[pl.ds(c * N_PAD + s * ROWS_PER_TILE, ROWS_PER_TILE)])

    zero_stripe()
    plsc.subcore_barrier()
    for src_tab, dst_tab, last in ((ego_hbm, t1, False), (t1, t2, False),
                                   (t2, t3, True)):
        edges(src_tab)
        plsc.subcore_barrier()
        writeback(dst_tab)
        if not last:
            zero_stripe()
        plsc.subcore_barrier()

    def fin(z, _):
        rr0 = c * N_PAD + s * ROWS_PER_TILE + z * ZCH
        pltpu.sync_copy(t1.at[pl.ds(rr0, ZCH)], r0)
        pltpu.sync_copy(t2.at[pl.ds(rr0, ZCH)], r1)
        pltpu.sync_copy(t3.at[pl.ds(rr0, ZCH)], r2)

        def m(i, _):
            for o in (0, 16):
                x = (r0[i, pl.ds(o, 16)] + r1[i, pl.ds(o, 16)]
                     + r2[i, pl.ds(o, 16)]) * jnp.float32(1.0 / 3.0)
                r0[i, pl.ds(o, 16)] = x
            return 0
        lax.fori_loop(0, ZCH, m, 0)
        pltpu.sync_copy(r0, out_hbm.at[pl.ds(rr0, ZCH)])
        return 0
    lax.fori_loop(0, NZC, fin, 0)


_run = pl.kernel(
    _body,
    out_type=jax.ShapeDtypeStruct((2 * N_PAD, H), jnp.float32),
    mesh=plsc.VectorSubcoreMesh(core_axis_name="c", subcore_axis_name="s"),
    compiler_params=pltpu.CompilerParams(use_tc_tiling_on_sc=False),
    scratch_types=[
        pltpu.HBM((2 * N_PAD, H), jnp.float32),
        pltpu.HBM((2 * N_PAD, H), jnp.float32),
        pltpu.HBM((2 * N_PAD, H), jnp.float32),
        pltpu.VMEM_SHARED((N_PAD, H), jnp.float32),
        pltpu.VMEM((GP, SUB), jnp.int32),
        pltpu.VMEM((GP, SUB), jnp.int32),
        pltpu.VMEM((GP, SUB), jnp.float32),
        pltpu.VMEM((GP, SUB), jnp.int32),
        pltpu.VMEM((GP, SUB), jnp.int32),
        pltpu.VMEM((GP, SUB), jnp.float32),
        pltpu.VMEM((SUB,), jnp.int32),
        pltpu.VMEM((SUB,), jnp.int32),
        pltpu.VMEM((SUB,), jnp.int32),
        pltpu.VMEM((SUB,), jnp.int32),
        pltpu.VMEM((SUB,), jnp.int32),
        pltpu.VMEM((SUB,), jnp.int32),
        pltpu.VMEM((SUB,), jnp.int32),
        pltpu.VMEM((SUB,), jnp.int32),
        pltpu.VMEM((SUB, H), jnp.float32),
        pltpu.VMEM((SUB, H), jnp.float32),
        pltpu.VMEM((SUB, H), jnp.float32),
        pltpu.VMEM((SUB, H), jnp.float32),
        pltpu.SemaphoreType.DMA,
        pltpu.SemaphoreType.DMA,
        pltpu.SemaphoreType.DMA,
        pltpu.SemaphoreType.DMA,
        pltpu.SemaphoreType.DMA,
        pltpu.SemaphoreType.DMA,
        pltpu.SemaphoreType.DMA,
        pltpu.SemaphoreType.DMA,
        pltpu.SemaphoreType.DMA,
    ],
)


def kernel(user_emb, item_emb, edge_val, edge_src, edge_dst):
    ego = jnp.concatenate([user_emb, item_emb], axis=0)
    rpad = jnp.zeros((N_PAD - N, H), jnp.float32)
    ego_flat = jnp.concatenate(
        [ego[:, :H], rpad, ego[:, H:], rpad], axis=0)
    e = edge_src.shape[0]
    pad = E_PAD - e
    src = jnp.pad(edge_src.astype(jnp.int32), (0, pad)).reshape(E_PAD // SUB, SUB)
    dst = jnp.pad(edge_dst.astype(jnp.int32), (0, pad)).reshape(E_PAD // SUB, SUB)
    val = jnp.pad(edge_val, (0, pad)).reshape(E_PAD // SUB, SUB)
    out = _run(ego_flat, src, dst, val)
    final = jnp.concatenate([out[:N], out[N_PAD:N_PAD + N]], axis=1)
    return final[:N_USERS], final[N_USERS:]
